# baseline (speedup 1.0000x reference)
import functools

import jax
import jax.numpy as jnp
from jax import lax
from jax.experimental import pallas as pl
from jax.experimental.pallas import tpu as pltpu
from jax.experimental.pallas import tpu_sc as plsc


def _build(B, T, D, dtype):
    info = plsc.get_sparse_core_info()
    NC = info.num_cores
    L = info.num_lanes
    mesh = plsc.VectorSubcoreMesh(core_axis_name="c", subcore_axis_name="s")

    @functools.partial(
        pl.kernel,
        mesh=mesh,
        out_type=jax.ShapeDtypeStruct((B, D), dtype),
        scratch_types=[
            pltpu.VMEM((T,), jnp.int32),
            pltpu.VMEM((D,), dtype),
        ],
    )
    def k(hidden_hbm, mask_hbm, out_hbm, mask_v, row_v):
        wid = lax.axis_index("s") * NC + lax.axis_index("c")

        @pl.when(wid < B)
        def _():
            b = wid
            pltpu.sync_copy(mask_hbm.at[b], mask_v)

            def body(i, acc):
                return acc + mask_v[pl.ds(i * L, L)]

            acc = lax.fori_loop(0, T // L, body, jnp.zeros((L,), jnp.int32))
            total = acc[0]
            for j in range(1, L):
                total = total + acc[j]
            slen = total - 1
            slen = jnp.maximum(jnp.minimum(slen, T - 1), 0)
            pltpu.sync_copy(hidden_hbm.at[b, slen], row_v)
            pltpu.sync_copy(row_v, out_hbm.at[b])

    return k


def kernel(hidden_states, attention_mask):
    B, T, D = hidden_states.shape
    k = _build(B, T, D, hidden_states.dtype)
    return k(hidden_states, attention_mask)


# speedup vs baseline: 1.0870x; 1.0870x over previous
import functools

import jax
import jax.numpy as jnp
from jax import lax
from jax.experimental import pallas as pl
from jax.experimental.pallas import tpu as pltpu
from jax.experimental.pallas import tpu_sc as plsc


def _build(B, T, D, dtype):
    info = plsc.get_sparse_core_info()
    NC = info.num_cores
    NS = info.num_subcores
    L = info.num_lanes

    bpc = max(1, B // NC)
    parts = NS // bpc
    chunk = T // parts

    mesh = plsc.VectorSubcoreMesh(core_axis_name="c", subcore_axis_name="s")

    @functools.partial(
        pl.kernel,
        mesh=mesh,
        out_type=jax.ShapeDtypeStruct((B, D), dtype),
        scratch_types=[
            pltpu.VMEM((chunk,), jnp.int32),
            pltpu.VMEM((L,), jnp.int32),
            pltpu.VMEM((parts, L), jnp.int32),
            pltpu.VMEM_SHARED((NS, L), jnp.int32),
            pltpu.VMEM((D,), dtype),
        ],
    )
    def k(hidden_hbm, mask_hbm, out_hbm, mask_v, part_v, gath_v, shared, row_v):
        c = lax.axis_index("c")
        s = lax.axis_index("s")
        lb = s // parts
        b = c * bpc + lb
        p = lax.rem(s, parts)
        active = b < B

        @pl.when(active)
        def _():
            pltpu.sync_copy(mask_hbm.at[b, pl.ds(p * chunk, chunk)], mask_v)

            def body(i, acc):
                return acc + mask_v[pl.ds(i * L, L)]

            acc = lax.fori_loop(0, chunk // L, body, jnp.zeros((L,), jnp.int32))
            part_v[...] = acc
            pltpu.sync_copy(part_v, shared.at[s])

        plsc.subcore_barrier()

        @pl.when(jnp.logical_and(active, p == 0))
        def _():
            pltpu.sync_copy(shared.at[pl.ds(s, parts)], gath_v)
            acc = gath_v[0]
            for q in range(1, parts):
                acc = acc + gath_v[q]
            total = acc[0]
            for j in range(1, L):
                total = total + acc[j]
            slen = total - 1
            slen = jnp.maximum(jnp.minimum(slen, T - 1), 0)
            pltpu.sync_copy(hidden_hbm.at[b, slen], row_v)
            pltpu.sync_copy(row_v, out_hbm.at[b])

    return k


def kernel(hidden_states, attention_mask):
    B, T, D = hidden_states.shape
    k = _build(B, T, D, hidden_states.dtype)
    return k(hidden_states, attention_mask)


# speedup vs baseline: 1.1038x; 1.0154x over previous
import functools

import jax
import jax.numpy as jnp
from jax import lax
from jax.experimental import pallas as pl
from jax.experimental.pallas import tpu as pltpu
from jax.experimental.pallas import tpu_sc as plsc


def _build(B, T, D, dtype):
    info = plsc.get_sparse_core_info()
    NC = info.num_cores
    NS = info.num_subcores
    L = info.num_lanes

    bpc = max(1, B // NC)
    parts = NS // bpc
    chunk = T // parts
    dchunk = D // parts

    mesh = plsc.VectorSubcoreMesh(core_axis_name="c", subcore_axis_name="s")

    @functools.partial(
        pl.kernel,
        mesh=mesh,
        out_type=jax.ShapeDtypeStruct((B, D), dtype),
        scratch_types=[
            pltpu.VMEM((chunk,), jnp.int32),
            pltpu.VMEM((L,), jnp.int32),
            pltpu.VMEM((parts, L), jnp.int32),
            pltpu.VMEM_SHARED((NS, L), jnp.int32),
            pltpu.VMEM((dchunk,), dtype),
        ],
    )
    def k(hidden_hbm, mask_hbm, out_hbm, mask_v, part_v, gath_v, shared, row_v):
        c = lax.axis_index("c")
        s = lax.axis_index("s")
        lb = s // parts
        b = c * bpc + lb
        p = lax.rem(s, parts)
        active = b < B

        @pl.when(active)
        def _():
            pltpu.sync_copy(mask_hbm.at[b, pl.ds(p * chunk, chunk)], mask_v)

            def body(i, acc):
                return acc + mask_v[pl.ds(i * L, L)]

            acc = lax.fori_loop(0, chunk // L, body, jnp.zeros((L,), jnp.int32))
            part_v[...] = acc
            pltpu.sync_copy(part_v, shared.at[s])

        plsc.subcore_barrier()

        @pl.when(active)
        def _():
            base = lb * parts
            pltpu.sync_copy(shared.at[pl.ds(base, parts)], gath_v)
            acc = gath_v[0]
            for q in range(1, parts):
                acc = acc + gath_v[q]
            total = acc[0]
            for j in range(1, L):
                total = total + acc[j]
            slen = total - 1
            slen = jnp.maximum(jnp.minimum(slen, T - 1), 0)
            off = p * dchunk
            pltpu.sync_copy(hidden_hbm.at[b, slen, pl.ds(off, dchunk)], row_v)
            pltpu.sync_copy(row_v, out_hbm.at[b, pl.ds(off, dchunk)])

    return k


def kernel(hidden_states, attention_mask):
    B, T, D = hidden_states.shape
    k = _build(B, T, D, hidden_states.dtype)
    return k(hidden_states, attention_mask)
